# Initial kernel scaffold; baseline (speedup 1.0000x reference)
import jax
import jax.numpy as jnp
from jax.experimental import pallas as pl


def kernel(x, embeddings):
    raise NotImplementedError("write your pallas kernel here")


# speedup vs baseline: 1.2779x; 1.2779x over previous
import functools

import jax
import jax.numpy as jnp
from jax import lax
from jax.experimental import pallas as pl
from jax.experimental.pallas import tpu as pltpu
from jax.experimental.pallas import tpu_sc as plsc

_D = 256
_K = 8192
_ROWS = 256
_KSUB = 1024
_LANES = 128
_BLOCK = 2736


def _segments(g):
    lo_k, hi_k = g * _LANES, (g + 1) * _LANES
    segs = []
    b0, b1 = lo_k // _BLOCK, (hi_k - 1) // _BLOCK
    for b in range(b0, b1 + 1):
        s = max(lo_k, b * _BLOCK) - lo_k
        e = min(hi_k, (b + 1) * _BLOCK) - lo_k
        segs.append((s, e, b))
    return segs


def _argmin_body(x_ref, e_ref, xx_ref, ee_ref, idx_ref):
    x = x_ref[...]
    xb = x.astype(jnp.bfloat16)
    xx = xx_ref[...]
    inf32 = jnp.float32(jnp.inf)
    imax = jnp.int32(2**31 - 1)
    acc_v = jnp.full((_ROWS, 1), inf32, jnp.float32)
    acc_i = jnp.zeros((_ROWS, 1), jnp.int32)
    run_v = jnp.full((_ROWS, _LANES), inf32, jnp.float32)
    run_i = jnp.zeros((_ROWS, _LANES), jnp.int32)
    for s in range(_K // _KSUB):
        e = e_ref[pl.ds(s * _KSUB, _KSUB), :]
        dot = lax.dot_general(xb, e.astype(jnp.bfloat16), (((1,), (1,)), ((), ())),
                              preferred_element_type=jnp.float32)
        for c in range(_KSUB // _LANES):
            g = s * (_KSUB // _LANES) + c
            kbase = g * _LANES
            ee_c = ee_ref[:, pl.ds(kbase, _LANES)]
            d = (xx + ee_c) - 2.0 * dot[:, c * _LANES:(c + 1) * _LANES]
            ki = lax.broadcasted_iota(jnp.int32, (_ROWS, _LANES), 1) + kbase
            for (slo, shi, b) in _segments(g):
                if slo == 0 and shi == _LANES:
                    dm = d
                else:
                    lane = lax.broadcasted_iota(jnp.int32, (_ROWS, _LANES), 1)
                    inseg = (lane >= slo) & (lane < shi)
                    dm = jnp.where(inseg, d, inf32)
                upd = dm < run_v
                run_i = jnp.where(upd, ki, run_i)
                run_v = jnp.where(upd, dm, run_v)
                if kbase + shi == min((b + 1) * _BLOCK, _K):
                    m = jnp.min(run_v, axis=1, keepdims=True)
                    cand = jnp.where(run_v == m, run_i, imax)
                    bi = jnp.min(cand, axis=1, keepdims=True)
                    better = m < acc_v
                    acc_v = jnp.where(better, m, acc_v)
                    acc_i = jnp.where(better, bi, acc_i)
                    acc_v = acc_v.astype(jnp.bfloat16).astype(jnp.float32)
                    run_v = jnp.full((_ROWS, _LANES), inf32, jnp.float32)
                    run_i = jnp.zeros((_ROWS, _LANES), jnp.int32)
    idx_ref[...] = jnp.broadcast_to(acc_i, (_ROWS, 8))


def _argmin_indices(flat_x, embeddings, xx, ee_row):
    n = flat_x.shape[0]
    idx8 = pl.pallas_call(
        _argmin_body,
        grid=(n // _ROWS,),
        in_specs=[
            pl.BlockSpec((_ROWS, _D), lambda i: (i, 0)),
            pl.BlockSpec((_K, _D), lambda i: (0, 0)),
            pl.BlockSpec((_ROWS, 1), lambda i: (i, 0)),
            pl.BlockSpec((1, _K), lambda i: (0, 0)),
        ],
        out_specs=pl.BlockSpec((_ROWS, 8), lambda i: (i, 0)),
        out_shape=jax.ShapeDtypeStruct((n, 8), jnp.int32),
    )(flat_x, embeddings, xx, ee_row)
    return idx8[:, 0]


def _make_gather(n):
    info = plsc.get_sparse_core_info()
    nw = info.num_cores * info.num_subcores
    chunk = 128
    b_per_w = n // nw
    nchunks = b_per_w // chunk
    mesh = plsc.VectorSubcoreMesh(core_axis_name="c", subcore_axis_name="s")

    @functools.partial(
        pl.kernel, mesh=mesh,
        out_type=jax.ShapeDtypeStruct((n, _D), jnp.float32),
        scratch_types=[
            pltpu.VMEM((chunk,), jnp.int32),
            pltpu.VMEM((chunk, _D), jnp.float32),
            pltpu.SemaphoreType.DMA,
        ],
    )
    def gather_kernel(table_hbm, idx_hbm, out_hbm, idx_v, rows_v, sem):
        wid = lax.axis_index("s") * info.num_cores + lax.axis_index("c")
        base = wid * b_per_w
        for c in range(nchunks):
            b = base + c * chunk
            pltpu.sync_copy(idx_hbm.at[pl.ds(b, chunk)], idx_v)
            pltpu.async_copy(table_hbm.at[idx_v], rows_v, sem).wait()
            pltpu.sync_copy(rows_v, out_hbm.at[pl.ds(b, chunk)])

    return gather_kernel


def kernel(x, embeddings):
    b, d, t = x.shape
    xp = jnp.transpose(x, (0, 2, 1))
    flat_x = xp.reshape(-1, d)
    xx = jnp.sum(flat_x ** 2, axis=1, keepdims=True)
    ee = jnp.sum(embeddings ** 2, axis=1)
    idx = _argmin_indices(flat_x, embeddings, xx, ee.reshape(1, _K))
    gathered = _make_gather(flat_x.shape[0])(embeddings, idx)
    quantized = gathered.reshape(xp.shape)
    quant = xp + jax.lax.stop_gradient(quantized - xp)
    out = jnp.transpose(quant, (0, 2, 1))
    return (out, out)
